# baseline (speedup 1.0000x reference)
import functools

import jax
import jax.numpy as jnp
from jax import lax
from jax.experimental import pallas as pl
from jax.experimental.pallas import tpu as pltpu
from jax.experimental.pallas import tpu_sc as plsc

_B, _L, _D = 4096, 200, 64
_DP = 128
_NC, _NS = 2, 16
_NW = _NC * _NS
_SEQ_W = _B // _NW
_IA, _IB = 128, _L - 128
_NBUF = 3


def _body(x_hbm, tok_hbm, pos_hbm, out_hbm, ia0, ia1, ia2, ib0, ib1, ib2,
          r0, r1, r2, pos_v, g0, g1, g2, s0, s1, s2):
    idxa = (ia0, ia1, ia2)
    idxb = (ib0, ib1, ib2)
    rows = (r0, r1, r2)
    sem_g = (g0, g1, g2)
    sem_st = (s0, s1, s2)
    wid = lax.axis_index("s") * _NC + lax.axis_index("c")
    seq0 = wid * _SEQ_W
    base = seq0 * _L
    pltpu.sync_copy(pos_hbm.at[pl.ds(0, _L)], pos_v)

    def fire_gather(b, j):
        r0_ = base + j * _L
        pltpu.sync_copy(x_hbm.at[pl.ds(r0_, _IA)], idxa[b])
        pltpu.sync_copy(x_hbm.at[pl.ds(r0_ + _IA, _IB)], idxb[b])
        pltpu.async_copy(tok_hbm.at[idxa[b]], rows[b].at[pl.ds(0, _IA)],
                         sem_g[b])
        pltpu.async_copy(tok_hbm.at[idxb[b]], rows[b].at[pl.ds(_IA, _IB)],
                         sem_g[b])

    def wait_store(b):
        pltpu.make_async_copy(rows[b], out_hbm.at[seq0], sem_st[b]).wait()

    def process(b, j):
        buf = rows[b]
        pltpu.make_async_copy(tok_hbm.at[idxa[b]], buf.at[pl.ds(0, _IA)],
                              sem_g[b]).wait()
        pltpu.make_async_copy(tok_hbm.at[idxb[b]], buf.at[pl.ds(_IA, _IB)],
                              sem_g[b]).wait()

        @plsc.parallel_loop(0, _L, step=1, unroll=8)
        def _add(r):
            for c in range(_D // 16):
                s = pl.ds(c * 16, 16)
                buf[r, s] = buf[r, s] + pos_v[r, s]

        pltpu.async_copy(buf, out_hbm.at[seq0 + j], sem_st[b])

    fire_gather(0, 0)
    fire_gather(1, 1)
    process(0, 0)
    fire_gather(2, 2)
    process(1, 1)
    wait_store(0)
    fire_gather(0, 3)
    process(2, 2)

    @pl.loop(3, _SEQ_W - 5, step=_NBUF)
    def _grp(g):
        for b in range(_NBUF):
            j = g + b
            fb = (b + 1) % _NBUF
            wait_store(fb)
            fire_gather(fb, j + 1)
            process(b, j)

    for j in range(_SEQ_W - 5, _SEQ_W):
        b = j % _NBUF
        if j + 1 < _SEQ_W:
            fb = (j + 1) % _NBUF
            wait_store(fb)
            fire_gather(fb, j + 1)
        process(b, j)
    for b in range(_NBUF):
        wait_store(b)


_V = 1000000
_TCHUNK = 512


def _fmt_body(tt_ref, out_ref):
    eye = jnp.eye(_D, dtype=jnp.float32)
    res = jax.lax.dot_general(tt_ref[...], eye, (((0,), (0,)), ((), ())),
                              preferred_element_type=jnp.float32)
    out_ref[...] = jnp.concatenate((res, res), axis=1)


_tc_fmt = pl.pallas_call(
    _fmt_body,
    out_shape=jax.ShapeDtypeStruct((_V, _DP), jnp.float32),
    grid=((_V + _TCHUNK - 1) // _TCHUNK,),
    in_specs=[pl.BlockSpec((_D, _TCHUNK), lambda i: (0, i))],
    out_specs=pl.BlockSpec((_TCHUNK, _DP), lambda i: (i, 0)),
)


_sc_embed = functools.partial(
    pl.kernel,
    mesh=plsc.VectorSubcoreMesh(core_axis_name="c", subcore_axis_name="s"),
    out_type=jax.ShapeDtypeStruct((_B, _L, _DP), jnp.float32),
    scratch_types=(
        [pltpu.VMEM((_IA,), jnp.int32) for _ in range(_NBUF)]
        + [pltpu.VMEM((_IB,), jnp.int32) for _ in range(_NBUF)]
        + [pltpu.VMEM((_L, _DP), jnp.float32) for _ in range(_NBUF)]
        + [pltpu.VMEM((_L, _D), jnp.float32)]
        + [pltpu.SemaphoreType.DMA for _ in range(2 * _NBUF)]
    ),
)(_body)


@jax.jit
def kernel(x, token_table, pos_table):
    xf = x.reshape(-1)
    tpad = _tc_fmt(token_table.T)
    out = _sc_embed(xf, tpad, pos_table)
    return out[..., :_D]


# speedup vs baseline: 2.0339x; 2.0339x over previous
import functools

import jax
import jax.numpy as jnp
from jax import lax
from jax.experimental import pallas as pl
from jax.experimental.pallas import tpu as pltpu
from jax.experimental.pallas import tpu_sc as plsc

_B, _L, _D = 4096, 200, 64
_DP = 128
_NC, _NS = 2, 16
_NW = _NC * _NS
_SEQ_W = _B // _NW
_IA, _IB = 128, _L - 128
_NBUF = 3


def _body(x_hbm, tok_hbm, pos_hbm, out_hbm, ia0, ia1, ia2, ib0, ib1, ib2,
          r0, r1, r2, pos_v, g0, g1, g2, s0, s1, s2):
    idxa = (ia0, ia1, ia2)
    idxb = (ib0, ib1, ib2)
    rows = (r0, r1, r2)
    sem_g = (g0, g1, g2)
    sem_st = (s0, s1, s2)
    wid = lax.axis_index("s") * _NC + lax.axis_index("c")
    seq0 = wid * _SEQ_W
    base = seq0 * _L
    pltpu.sync_copy(pos_hbm.at[pl.ds(0, _L)], pos_v)

    def fire_gather(b, j):
        r0_ = base + j * _L
        pltpu.sync_copy(x_hbm.at[pl.ds(r0_, _IA)], idxa[b])
        pltpu.sync_copy(x_hbm.at[pl.ds(r0_ + _IA, _IB)], idxb[b])
        pltpu.async_copy(tok_hbm.at[idxa[b]], rows[b].at[pl.ds(0, _IA)],
                         sem_g[b])
        pltpu.async_copy(tok_hbm.at[idxb[b]], rows[b].at[pl.ds(_IA, _IB)],
                         sem_g[b])

    def wait_store(b):
        pltpu.make_async_copy(rows[b], out_hbm.at[seq0], sem_st[b]).wait()

    def process(b, j):
        buf = rows[b]
        pltpu.make_async_copy(tok_hbm.at[idxa[b]], buf.at[pl.ds(0, _IA)],
                              sem_g[b]).wait()
        pltpu.make_async_copy(tok_hbm.at[idxb[b]], buf.at[pl.ds(_IA, _IB)],
                              sem_g[b]).wait()

        @plsc.parallel_loop(0, _L, step=1, unroll=8)
        def _add(r):
            for c in range(_D // 16):
                s = pl.ds(c * 16, 16)
                buf[r, s] = buf[r, s] + pos_v[r, s]

        pltpu.async_copy(buf, out_hbm.at[seq0 + j], sem_st[b])

    fire_gather(0, 0)
    fire_gather(1, 1)
    process(0, 0)
    fire_gather(2, 2)
    process(1, 1)
    wait_store(0)
    fire_gather(0, 3)
    process(2, 2)

    @pl.loop(3, _SEQ_W - 5, step=_NBUF)
    def _grp(g):
        for b in range(_NBUF):
            j = g + b
            fb = (b + 1) % _NBUF
            wait_store(fb)
            fire_gather(fb, j + 1)
            process(b, j)

    for j in range(_SEQ_W - 5, _SEQ_W):
        b = j % _NBUF
        if j + 1 < _SEQ_W:
            fb = (j + 1) % _NBUF
            wait_store(fb)
            fire_gather(fb, j + 1)
        process(b, j)
    for b in range(_NBUF):
        wait_store(b)


_V = 1000000
_TCHUNK = 4096


def _fmt_body(tt_ref, out_ref):
    eye = jnp.eye(_D, dtype=jnp.float32)
    res = jax.lax.dot_general(tt_ref[...], eye, (((0,), (0,)), ((), ())),
                              preferred_element_type=jnp.float32)
    out_ref[...] = jnp.concatenate((res, res), axis=1)


_tc_fmt = pl.pallas_call(
    _fmt_body,
    out_shape=jax.ShapeDtypeStruct((_V, _DP), jnp.float32),
    grid=((_V + _TCHUNK - 1) // _TCHUNK,),
    in_specs=[pl.BlockSpec((_D, _TCHUNK), lambda i: (0, i))],
    out_specs=pl.BlockSpec((_TCHUNK, _DP), lambda i: (i, 0)),
)


_sc_embed = functools.partial(
    pl.kernel,
    mesh=plsc.VectorSubcoreMesh(core_axis_name="c", subcore_axis_name="s"),
    out_type=jax.ShapeDtypeStruct((_B, _L, _DP), jnp.float32),
    scratch_types=(
        [pltpu.VMEM((_IA,), jnp.int32) for _ in range(_NBUF)]
        + [pltpu.VMEM((_IB,), jnp.int32) for _ in range(_NBUF)]
        + [pltpu.VMEM((_L, _DP), jnp.float32) for _ in range(_NBUF)]
        + [pltpu.VMEM((_L, _D), jnp.float32)]
        + [pltpu.SemaphoreType.DMA for _ in range(2 * _NBUF)]
    ),
)(_body)


@jax.jit
def kernel(x, token_table, pos_table):
    xf = x.reshape(-1)
    tpad = _tc_fmt(token_table.T)
    out = _sc_embed(xf, tpad, pos_table)
    return out[..., :_D]


# speedup vs baseline: 2.2969x; 1.1293x over previous
import functools

import jax
import jax.numpy as jnp
from jax import lax
from jax.experimental import pallas as pl
from jax.experimental.pallas import tpu as pltpu
from jax.experimental.pallas import tpu_sc as plsc

_B, _L, _D = 4096, 200, 64
_DP = 128
_NC, _NS = 2, 16
_NW = _NC * _NS
_SEQ_W = _B // _NW
_IA, _IB = 128, _L - 128
_NBUF = 3


def _body(x_hbm, tok_hbm, pos_hbm, out_hbm, ia0, ia1, ia2, ib0, ib1, ib2,
          r0, r1, r2, pos_v, g0, g1, g2, s0, s1, s2):
    idxa = (ia0, ia1, ia2)
    idxb = (ib0, ib1, ib2)
    rows = (r0, r1, r2)
    sem_g = (g0, g1, g2)
    sem_st = (s0, s1, s2)
    wid = lax.axis_index("s") * _NC + lax.axis_index("c")
    seq0 = wid * _SEQ_W
    base = seq0 * _L
    pltpu.sync_copy(pos_hbm.at[pl.ds(0, _L)], pos_v)

    def fire_gather(b, j):
        r0_ = base + j * _L
        pltpu.sync_copy(x_hbm.at[pl.ds(r0_, _IA)], idxa[b])
        pltpu.sync_copy(x_hbm.at[pl.ds(r0_ + _IA, _IB)], idxb[b])
        pltpu.async_copy(tok_hbm.at[idxa[b]], rows[b].at[pl.ds(0, _IA)],
                         sem_g[b])
        pltpu.async_copy(tok_hbm.at[idxb[b]], rows[b].at[pl.ds(_IA, _IB)],
                         sem_g[b])

    def wait_store(b):
        pltpu.make_async_copy(rows[b], out_hbm.at[seq0], sem_st[b]).wait()

    def process(b, j):
        buf = rows[b]
        pltpu.make_async_copy(tok_hbm.at[idxa[b]], buf.at[pl.ds(0, _IA)],
                              sem_g[b]).wait()
        pltpu.make_async_copy(tok_hbm.at[idxb[b]], buf.at[pl.ds(_IA, _IB)],
                              sem_g[b]).wait()

        @plsc.parallel_loop(0, _L, step=1, unroll=8)
        def _add(r):
            for c in range(_D // 16):
                s = pl.ds(c * 16, 16)
                buf[r, s] = buf[r, s] + pos_v[r, s]

        pltpu.async_copy(buf, out_hbm.at[seq0 + j], sem_st[b])

    fire_gather(0, 0)
    fire_gather(1, 1)
    process(0, 0)
    fire_gather(2, 2)
    process(1, 1)
    wait_store(0)
    fire_gather(0, 3)
    process(2, 2)

    @pl.loop(3, _SEQ_W - 5, step=_NBUF)
    def _grp(g):
        for b in range(_NBUF):
            j = g + b
            fb = (b + 1) % _NBUF
            wait_store(fb)
            fire_gather(fb, j + 1)
            process(b, j)

    for j in range(_SEQ_W - 5, _SEQ_W):
        b = j % _NBUF
        if j + 1 < _SEQ_W:
            fb = (j + 1) % _NBUF
            wait_store(fb)
            fire_gather(fb, j + 1)
        process(b, j)
    for b in range(_NBUF):
        wait_store(b)


_V = 1000000
_TCHUNK = 16384


def _fmt_body(tt_ref, out_ref):
    eye = jnp.eye(_D, dtype=jnp.float32)
    res = jax.lax.dot_general(tt_ref[...], eye, (((0,), (0,)), ((), ())),
                              preferred_element_type=jnp.float32)
    out_ref[...] = jnp.concatenate((res, res), axis=1)


_tc_fmt = pl.pallas_call(
    _fmt_body,
    out_shape=jax.ShapeDtypeStruct((_V, _DP), jnp.float32),
    grid=((_V + _TCHUNK - 1) // _TCHUNK,),
    in_specs=[pl.BlockSpec((_D, _TCHUNK), lambda i: (0, i))],
    out_specs=pl.BlockSpec((_TCHUNK, _DP), lambda i: (i, 0)),
)


_sc_embed = functools.partial(
    pl.kernel,
    mesh=plsc.VectorSubcoreMesh(core_axis_name="c", subcore_axis_name="s"),
    out_type=jax.ShapeDtypeStruct((_B, _L, _DP), jnp.float32),
    scratch_types=(
        [pltpu.VMEM((_IA,), jnp.int32) for _ in range(_NBUF)]
        + [pltpu.VMEM((_IB,), jnp.int32) for _ in range(_NBUF)]
        + [pltpu.VMEM((_L, _DP), jnp.float32) for _ in range(_NBUF)]
        + [pltpu.VMEM((_L, _D), jnp.float32)]
        + [pltpu.SemaphoreType.DMA for _ in range(2 * _NBUF)]
    ),
)(_body)


@jax.jit
def kernel(x, token_table, pos_table):
    xf = x.reshape(-1)
    tpad = _tc_fmt(token_table.T)
    out = _sc_embed(xf, tpad, pos_table)
    return out[..., :_D]
